# Initial kernel scaffold; baseline (speedup 1.0000x reference)
import jax
import jax.numpy as jnp
from jax.experimental import pallas as pl


def kernel(node_feats, batch_idx, w1, b1, w2, b2):
    raise NotImplementedError("write your pallas kernel here")


# speedup vs baseline: 1.2618x; 1.2618x over previous
import functools

import jax
import jax.numpy as jnp
from jax.experimental import pallas as pl
from jax.experimental.pallas import tpu as pltpu

LANE = 128


def _pool_body(graph_ids_ref, batch_ref, x_ref, partial_ref):
    t = pl.program_id(1)

    @pl.when(t == 0)
    def _():
        partial_ref[...] = jnp.zeros_like(partial_ref)

    onehot = (graph_ids_ref[...] == batch_ref[...]).astype(jnp.bfloat16)
    x = x_ref[...].astype(jnp.bfloat16)
    partial_ref[0] = partial_ref[0] + jnp.dot(
        onehot, x, preferred_element_type=jnp.float32)


def _mlp_body(partial_ref, w1_ref, b1_ref, w2_ref, b2_ref, out_ref):
    pooled = jnp.sum(partial_ref[...], axis=0)
    h = jnp.dot(pooled, w1_ref[...],
                preferred_element_type=jnp.float32) + b1_ref[...]
    h = jnp.where(h > 0, h, 0.01 * h)
    out = jnp.dot(h, w2_ref[...],
                  preferred_element_type=jnp.float32) + b2_ref[...]
    out_ref[...] = out.astype(out_ref.dtype)


@functools.partial(jax.jit, static_argnames=("num_graphs",))
def _forward(node_feats, batch_idx, w1, b1, w2, b2, *, num_graphs):
    n_nodes, d_in = node_feats.shape
    out_dim = w2.shape[1]

    num_splits = 2
    tile_n = 8192
    while n_nodes % (num_splits * tile_n) != 0:
        tile_n //= 2
    tiles_per_split = n_nodes // (num_splits * tile_n)

    batch_lane = batch_idx.reshape(1, n_nodes).astype(jnp.int32)
    graph_ids = jnp.arange(num_graphs, dtype=jnp.int32).reshape(num_graphs, 1)

    partials = pl.pallas_call(
        _pool_body,
        out_shape=jax.ShapeDtypeStruct((num_splits, num_graphs, d_in),
                                       jnp.float32),
        grid=(num_splits, tiles_per_split),
        in_specs=[
            pl.BlockSpec((num_graphs, 1), lambda s, t: (0, 0)),
            pl.BlockSpec((1, tile_n),
                         lambda s, t: (0, s * tiles_per_split + t)),
            pl.BlockSpec((tile_n, d_in),
                         lambda s, t: (s * tiles_per_split + t, 0)),
        ],
        out_specs=pl.BlockSpec((1, num_graphs, d_in), lambda s, t: (s, 0, 0)),
        compiler_params=pltpu.CompilerParams(
            dimension_semantics=("parallel", "arbitrary"),
            vmem_limit_bytes=48 * 1024 * 1024,
        ),
    )(graph_ids, batch_lane, node_feats)

    return pl.pallas_call(
        _mlp_body,
        out_shape=jax.ShapeDtypeStruct((num_graphs, out_dim), jnp.float32),
        grid=(1,),
        in_specs=[
            pl.BlockSpec((num_splits, num_graphs, d_in), lambda i: (0, 0, 0)),
            pl.BlockSpec(w1.shape, lambda i: (0, 0)),
            pl.BlockSpec((1, w1.shape[1]), lambda i: (0, 0)),
            pl.BlockSpec(w2.shape, lambda i: (0, 0)),
            pl.BlockSpec((1, out_dim), lambda i: (0, 0)),
        ],
        out_specs=pl.BlockSpec((num_graphs, out_dim), lambda i: (0, 0)),
        compiler_params=pltpu.CompilerParams(
            dimension_semantics=("arbitrary",),
        ),
    )(partials, w1, b1.reshape(1, -1), w2, b2.reshape(1, -1))


def kernel(node_feats, batch_idx, w1, b1, w2, b2):
    return _forward(node_feats, batch_idx, w1, b1, w2, b2, num_graphs=256)


# speedup vs baseline: 1.4148x; 1.1212x over previous
import functools

import jax
import jax.numpy as jnp
from jax.experimental import pallas as pl
from jax.experimental.pallas import tpu as pltpu

LANE = 128


def _pool_body(graph_ids_ref, batch_ref, x_ref, partial_ref):
    t = pl.program_id(1)

    @pl.when(t == 0)
    def _():
        partial_ref[...] = jnp.zeros_like(partial_ref)

    onehot = (graph_ids_ref[...] == batch_ref[...]).astype(jnp.bfloat16)
    x = x_ref[...].astype(jnp.bfloat16)
    partial_ref[0] = partial_ref[0] + jnp.dot(
        onehot, x, preferred_element_type=jnp.float32)


def _mlp_body(partial_ref, w1_ref, b1_ref, w2_ref, b2_ref, out_ref):
    pooled = jnp.sum(partial_ref[...], axis=0)
    h = jnp.dot(pooled, w1_ref[...],
                preferred_element_type=jnp.float32) + b1_ref[...]
    h = jnp.where(h > 0, h, 0.01 * h)
    out = jnp.dot(h, w2_ref[...],
                  preferred_element_type=jnp.float32) + b2_ref[...]
    out_ref[...] = out.astype(out_ref.dtype)


@functools.partial(jax.jit, static_argnames=("num_graphs",))
def _forward(node_feats, batch_idx, w1, b1, w2, b2, *, num_graphs):
    n_nodes, d_in = node_feats.shape
    out_dim = w2.shape[1]

    num_splits = 2
    tile_n = 16384
    while n_nodes % (num_splits * tile_n) != 0:
        tile_n //= 2
    tiles_per_split = n_nodes // (num_splits * tile_n)

    batch_lane = batch_idx.reshape(1, n_nodes).astype(jnp.int32)
    graph_ids = jnp.arange(num_graphs, dtype=jnp.int32).reshape(num_graphs, 1)

    partials = pl.pallas_call(
        _pool_body,
        out_shape=jax.ShapeDtypeStruct((num_splits, num_graphs, d_in),
                                       jnp.float32),
        grid=(num_splits, tiles_per_split),
        in_specs=[
            pl.BlockSpec((num_graphs, 1), lambda s, t: (0, 0)),
            pl.BlockSpec((1, tile_n),
                         lambda s, t: (0, s * tiles_per_split + t)),
            pl.BlockSpec((tile_n, d_in),
                         lambda s, t: (s * tiles_per_split + t, 0)),
        ],
        out_specs=pl.BlockSpec((1, num_graphs, d_in), lambda s, t: (s, 0, 0)),
        compiler_params=pltpu.CompilerParams(
            dimension_semantics=("parallel", "arbitrary"),
            vmem_limit_bytes=48 * 1024 * 1024,
        ),
    )(graph_ids, batch_lane, node_feats)

    return pl.pallas_call(
        _mlp_body,
        out_shape=jax.ShapeDtypeStruct((num_graphs, out_dim), jnp.float32),
        grid=(1,),
        in_specs=[
            pl.BlockSpec((num_splits, num_graphs, d_in), lambda i: (0, 0, 0)),
            pl.BlockSpec(w1.shape, lambda i: (0, 0)),
            pl.BlockSpec((1, w1.shape[1]), lambda i: (0, 0)),
            pl.BlockSpec(w2.shape, lambda i: (0, 0)),
            pl.BlockSpec((1, out_dim), lambda i: (0, 0)),
        ],
        out_specs=pl.BlockSpec((num_graphs, out_dim), lambda i: (0, 0)),
        compiler_params=pltpu.CompilerParams(
            dimension_semantics=("arbitrary",),
        ),
    )(partials, w1, b1.reshape(1, -1), w2, b2.reshape(1, -1))


def kernel(node_feats, batch_idx, w1, b1, w2, b2):
    return _forward(node_feats, batch_idx, w1, b1, w2, b2, num_graphs=256)


# speedup vs baseline: 1.4234x; 1.0061x over previous
import functools

import jax
import jax.numpy as jnp
from jax.experimental import pallas as pl
from jax.experimental.pallas import tpu as pltpu

LANE = 128


def _pool_body(graph_ids_ref, batch_ref, x_ref, partial_ref):
    t = pl.program_id(1)

    @pl.when(t == 0)
    def _():
        partial_ref[...] = jnp.zeros_like(partial_ref)

    onehot = (graph_ids_ref[...] == batch_ref[...]).astype(jnp.bfloat16)
    x = x_ref[...].astype(jnp.bfloat16)
    partial_ref[0] = partial_ref[0] + jnp.dot(
        onehot, x, preferred_element_type=jnp.float32)


def _mlp_body(partial_ref, w1_ref, b1_ref, w2_ref, b2_ref, out_ref):
    pooled = jnp.sum(partial_ref[...], axis=0)
    h = jnp.dot(pooled, w1_ref[...],
                preferred_element_type=jnp.float32) + b1_ref[...]
    h = jnp.where(h > 0, h, 0.01 * h)
    out = jnp.dot(h, w2_ref[...],
                  preferred_element_type=jnp.float32) + b2_ref[...]
    out_ref[...] = out.astype(out_ref.dtype)


@functools.partial(jax.jit, static_argnames=("num_graphs",))
def _forward(node_feats, batch_idx, w1, b1, w2, b2, *, num_graphs):
    n_nodes, d_in = node_feats.shape
    out_dim = w2.shape[1]

    num_splits = 1
    tile_n = 16384
    while n_nodes % (num_splits * tile_n) != 0:
        tile_n //= 2
    tiles_per_split = n_nodes // (num_splits * tile_n)

    batch_lane = batch_idx.reshape(1, n_nodes).astype(jnp.int32)
    graph_ids = jnp.arange(num_graphs, dtype=jnp.int32).reshape(num_graphs, 1)

    partials = pl.pallas_call(
        _pool_body,
        out_shape=jax.ShapeDtypeStruct((num_splits, num_graphs, d_in),
                                       jnp.float32),
        grid=(num_splits, tiles_per_split),
        in_specs=[
            pl.BlockSpec((num_graphs, 1), lambda s, t: (0, 0)),
            pl.BlockSpec((1, tile_n),
                         lambda s, t: (0, s * tiles_per_split + t)),
            pl.BlockSpec((tile_n, d_in),
                         lambda s, t: (s * tiles_per_split + t, 0)),
        ],
        out_specs=pl.BlockSpec((1, num_graphs, d_in), lambda s, t: (s, 0, 0)),
        compiler_params=pltpu.CompilerParams(
            dimension_semantics=("parallel", "arbitrary"),
            vmem_limit_bytes=48 * 1024 * 1024,
        ),
    )(graph_ids, batch_lane, node_feats)

    return pl.pallas_call(
        _mlp_body,
        out_shape=jax.ShapeDtypeStruct((num_graphs, out_dim), jnp.float32),
        grid=(1,),
        in_specs=[
            pl.BlockSpec((num_splits, num_graphs, d_in), lambda i: (0, 0, 0)),
            pl.BlockSpec(w1.shape, lambda i: (0, 0)),
            pl.BlockSpec((1, w1.shape[1]), lambda i: (0, 0)),
            pl.BlockSpec(w2.shape, lambda i: (0, 0)),
            pl.BlockSpec((1, out_dim), lambda i: (0, 0)),
        ],
        out_specs=pl.BlockSpec((num_graphs, out_dim), lambda i: (0, 0)),
        compiler_params=pltpu.CompilerParams(
            dimension_semantics=("arbitrary",),
        ),
    )(partials, w1, b1.reshape(1, -1), w2, b2.reshape(1, -1))


def kernel(node_feats, batch_idx, w1, b1, w2, b2):
    return _forward(node_feats, batch_idx, w1, b1, w2, b2, num_graphs=256)
